# Initial kernel scaffold; baseline (speedup 1.0000x reference)
import jax
import jax.numpy as jnp
from jax.experimental import pallas as pl


def kernel(x, edge_index, edge_weight, W0, b0, W1, b1):
    raise NotImplementedError("write your pallas kernel here")


# speedup vs baseline: 9.0197x; 9.0197x over previous
import functools

import jax
import jax.numpy as jnp
from jax import lax
from jax.experimental import pallas as pl
from jax.experimental.pallas import tpu as pltpu
from jax.experimental.pallas import tpu_sc as plsc

N = 10000
E = 320000
D = 128
NC = 2
NS = 16
NW = NC * NS
K = 128
EPW = 10240
E_PAD = EPW * NW
NCHUNK = EPW // K

_MESH = plsc.VectorSubcoreMesh(core_axis_name="c", subcore_axis_name="s")
_SC_PARAMS = pltpu.CompilerParams(needs_layout_passes=False)


@functools.partial(
    pl.kernel,
    out_type=jax.ShapeDtypeStruct((NC, N), jnp.float32),
    mesh=_MESH,
    scratch_types=[
        pltpu.VMEM((K,), jnp.int32),
        pltpu.VMEM((K,), jnp.float32),
        pltpu.VMEM_SHARED((N,), jnp.float32),
    ],
    compiler_params=_SC_PARAMS,
)
def _deg_kernel(col_hbm, ew_hbm, zn_hbm, degp_hbm, coli_v, ew_v, deg_sh):
    c = lax.axis_index("c")
    s = lax.axis_index("s")
    wid = s * NC + c

    @pl.when(s == 0)
    def _():
        pltpu.sync_copy(zn_hbm, deg_sh)

    plsc.subcore_barrier()

    def chunk(ch, carry):
        base = wid * EPW + ch * K
        pltpu.sync_copy(col_hbm.at[pl.ds(base, K)], coli_v)
        pltpu.sync_copy(ew_hbm.at[pl.ds(base, K)], ew_v)
        pltpu.sync_copy(ew_v, deg_sh.at[coli_v], add=True)
        return carry

    lax.fori_loop(0, NCHUNK, chunk, 0)
    plsc.subcore_barrier()

    @pl.when(s == 0)
    def _():
        pltpu.sync_copy(deg_sh, degp_hbm.at[c])


_MB = 1000


def _mid_body(x_ref, w_ref, degp_ref, hp_ref, dinv_ref):
    deg = jnp.sum(degp_ref[...], axis=1) + 1.0
    dinv = lax.rsqrt(deg)
    h = jnp.dot(x_ref[...], w_ref[...], preferred_element_type=jnp.float32)
    hp_ref[...] = h * dinv[:, None]
    dinv_ref[...] = dinv[:, None]


def _mid_call(x, w1, degp):
    return pl.pallas_call(
        _mid_body,
        grid=(N // _MB,),
        in_specs=[
            pl.BlockSpec((_MB, D), lambda i: (i, 0)),
            pl.BlockSpec((D, D), lambda i: (0, 0)),
            pl.BlockSpec((_MB, NC), lambda i: (i, 0)),
        ],
        out_specs=[
            pl.BlockSpec((_MB, D), lambda i: (i, 0)),
            pl.BlockSpec((_MB, 1), lambda i: (i, 0)),
        ],
        out_shape=[
            jax.ShapeDtypeStruct((N, D), jnp.float32),
            jax.ShapeDtypeStruct((N, 1), jnp.float32),
        ],
    )(x, w1, degp)


@functools.partial(
    pl.kernel,
    out_type=jax.ShapeDtypeStruct((NC, N, D), jnp.float32),
    mesh=_MESH,
    scratch_types=[
        pltpu.VMEM((K,), jnp.int32),
        pltpu.VMEM((K,), jnp.int32),
        pltpu.VMEM((K,), jnp.float32),
        pltpu.VMEM((K, D), jnp.float32),
        pltpu.VMEM_SHARED((N, D), jnp.float32),
        pltpu.SemaphoreType.DMA,
    ],
    compiler_params=_SC_PARAMS,
)
def _scatter_kernel(row_hbm, col_hbm, ew_hbm, hp_hbm, znd_hbm, accp_hbm,
                    rowi_v, coli_v, ew_v, rows_v, acc_sh, sem):
    c = lax.axis_index("c")
    s = lax.axis_index("s")
    wid = s * NC + c

    @pl.when(s == 0)
    def _():
        pltpu.sync_copy(znd_hbm, acc_sh)

    plsc.subcore_barrier()

    def chunk(ch, carry):
        base = wid * EPW + ch * K
        pltpu.sync_copy(row_hbm.at[pl.ds(base, K)], rowi_v)
        pltpu.sync_copy(col_hbm.at[pl.ds(base, K)], coli_v)
        pltpu.sync_copy(ew_hbm.at[pl.ds(base, K)], ew_v)
        pltpu.async_copy(hp_hbm.at[rowi_v], rows_v, sem).wait()

        def edge(e, inner):
            ewb = plsc.load_gather(ew_v, [jnp.full((16,), e, jnp.int32)])
            for f in range(D // 16):
                sl = pl.ds(f * 16, 16)
                rows_v[e, sl] = rows_v[e, sl] * ewb
            return inner

        lax.fori_loop(0, K, edge, 0)
        pltpu.sync_copy(rows_v, acc_sh.at[coli_v], add=True)
        return carry

    lax.fori_loop(0, NCHUNK, chunk, 0)
    plsc.subcore_barrier()

    @pl.when(s == 0)
    def _():
        pltpu.sync_copy(acc_sh, accp_hbm.at[c])


def _final_body(accp_ref, hp_ref, dinv_ref, b_ref, out_ref):
    sacc = accp_ref[0] + accp_ref[1] + hp_ref[...]
    y = dinv_ref[...] * sacc + b_ref[...][None, :]
    out_ref[...] = jnp.where(y > 0, y, jnp.exp(y) - 1.0)


def _final_call(accp, hp, dinv, b1):
    return pl.pallas_call(
        _final_body,
        grid=(N // _MB,),
        in_specs=[
            pl.BlockSpec((NC, _MB, D), lambda i: (0, i, 0)),
            pl.BlockSpec((_MB, D), lambda i: (i, 0)),
            pl.BlockSpec((_MB, 1), lambda i: (i, 0)),
            pl.BlockSpec((D,), lambda i: (0,)),
        ],
        out_specs=pl.BlockSpec((_MB, D), lambda i: (i, 0)),
        out_shape=jax.ShapeDtypeStruct((N, D), jnp.float32),
    )(accp, hp, dinv, b1)


def kernel(x, edge_index, edge_weight, W0, b0, W1, b1):
    row = edge_index[0]
    col = edge_index[1]
    pad = E_PAD - E
    zi = jnp.zeros((pad,), jnp.int32)
    rowp = jnp.concatenate([row, zi])
    colp = jnp.concatenate([col, zi])
    ewp = jnp.concatenate([edge_weight, jnp.zeros((pad,), jnp.float32)])
    zn = jnp.zeros((N,), jnp.float32)
    znd = jnp.zeros((N, D), jnp.float32)

    degp = _deg_kernel(colp, ewp, zn)
    hp, dinv = _mid_call(x, W1, degp.T)
    accp = _scatter_kernel(rowp, colp, ewp, hp, znd)
    return _final_call(accp, hp, dinv, b1)


# speedup vs baseline: 24.2885x; 2.6928x over previous
import functools

import jax
import jax.numpy as jnp
from jax import lax
from jax.experimental import pallas as pl
from jax.experimental.pallas import tpu as pltpu
from jax.experimental.pallas import tpu_sc as plsc

N = 10000
E = 320000
D = 128
NC = 2
NS = 16
NW = NC * NS
K = 80
NCHUNK = 126
EPW = NCHUNK * K
E_PAD = EPW * NW
NPAIR = NCHUNK // 2
NROWS = 624

_MESH = plsc.VectorSubcoreMesh(core_axis_name="c", subcore_axis_name="s")
_SC_PARAMS = pltpu.CompilerParams(needs_layout_passes=False)


_DEGW = 8


@functools.partial(
    pl.kernel,
    out_type=jax.ShapeDtypeStruct((NC, N), jnp.float32),
    mesh=_MESH,
    scratch_types=[
        pltpu.VMEM((NCHUNK, K), jnp.int32),
        pltpu.VMEM((NCHUNK, K), jnp.float32),
        pltpu.VMEM_SHARED((N,), jnp.float32),
        pltpu.SemaphoreType.DMA,
    ],
    compiler_params=_SC_PARAMS,
)
def _deg_kernel(col_hbm, ew_hbm, zn_hbm, degp_hbm, col_v, ew_v, deg_sh, sem):
    c = lax.axis_index("c")
    s = lax.axis_index("s")
    wid = s * NC + c

    @pl.when(s == 0)
    def _():
        pltpu.sync_copy(zn_hbm, deg_sh)

    pltpu.sync_copy(col_hbm.at[wid], col_v)
    pltpu.sync_copy(ew_hbm.at[wid], ew_v)
    plsc.subcore_barrier()

    def drain_one():
        pltpu.make_async_copy(ew_v.at[0], deg_sh.at[col_v.at[0]], sem).wait()

    def fire(ch, carry):
        pltpu.async_copy(ew_v.at[ch], deg_sh.at[col_v.at[ch]], sem, add=True)

        @pl.when(ch >= _DEGW)
        def _():
            drain_one()

        return carry

    lax.fori_loop(0, NCHUNK, fire, 0)

    def drain(ch, carry):
        drain_one()
        return carry

    lax.fori_loop(0, _DEGW, drain, 0)
    plsc.subcore_barrier()

    @pl.when(s == 0)
    def _():
        pltpu.sync_copy(deg_sh, degp_hbm.at[c])


_MB = 1000


def _mid_body(x_ref, w_ref, degp_ref, hp_ref, dinv_ref):
    deg = jnp.sum(degp_ref[...], axis=1) + 1.0
    dinv = lax.rsqrt(deg)
    h = jnp.dot(x_ref[...], w_ref[...], preferred_element_type=jnp.float32)
    hp_ref[...] = h * dinv[:, None]
    dinv_ref[...] = dinv[:, None]


def _mid_call(x, w1, degp_t):
    return pl.pallas_call(
        _mid_body,
        grid=(N // _MB,),
        in_specs=[
            pl.BlockSpec((_MB, D), lambda i: (i, 0)),
            pl.BlockSpec((D, D), lambda i: (0, 0)),
            pl.BlockSpec((_MB, NC), lambda i: (i, 0)),
        ],
        out_specs=[
            pl.BlockSpec((_MB, D), lambda i: (i, 0)),
            pl.BlockSpec((_MB, 1), lambda i: (i, 0)),
        ],
        out_shape=[
            jax.ShapeDtypeStruct((N, D), jnp.float32),
            jax.ShapeDtypeStruct((N, 1), jnp.float32),
        ],
    )(x, w1, degp_t)


@functools.partial(
    pl.kernel,
    out_type=jax.ShapeDtypeStruct((NC, N, D), jnp.float32),
    mesh=_MESH,
    scratch_types=[
        pltpu.VMEM((18, K), jnp.int32),
        pltpu.VMEM((K, D), jnp.float32),
        pltpu.VMEM((K, D), jnp.float32),
        pltpu.VMEM((K, D), jnp.float32),
        pltpu.VMEM((K, D), jnp.float32),
        pltpu.VMEM_SHARED((N, D), jnp.float32),
        pltpu.SemaphoreType.DMA,
        pltpu.SemaphoreType.DMA,
        pltpu.SemaphoreType.DMA,
        pltpu.SemaphoreType.DMA,
        pltpu.SemaphoreType.DMA,
        pltpu.SemaphoreType.DMA,
    ],
    compiler_params=_SC_PARAMS,
)
def _scatter_kernel(idx_hbm, hp_hbm, znd_hbm, accp_hbm,
                    idxb, ga, gb, sa, sb, acc_sh,
                    semga, semgb, semsa, semsb, semia, semib):
    c = lax.axis_index("c")
    s = lax.axis_index("s")
    wid = s * NC + c

    pltpu.sync_copy(idx_hbm.at[wid, 0], idxb.at[pl.ds(0, 3)])
    pltpu.sync_copy(idx_hbm.at[wid, 1], idxb.at[pl.ds(3, 3)])
    pltpu.sync_copy(idx_hbm.at[wid, 2], idxb.at[pl.ds(6, 3)])

    @pl.when(s < NS - 1)
    def _():
        pltpu.sync_copy(znd_hbm.at[pl.ds(s * NROWS, NROWS)],
                        acc_sh.at[pl.ds(s * NROWS, NROWS)])

    @pl.when(s == NS - 1)
    def _():
        pltpu.sync_copy(znd_hbm.at[pl.ds((NS - 1) * NROWS, N - (NS - 1) * NROWS)],
                        acc_sh.at[pl.ds((NS - 1) * NROWS, N - (NS - 1) * NROWS)])

    plsc.subcore_barrier()

    def scale(gbuf, sbuf, ewrow):
        @plsc.parallel_loop(0, K, unroll=4)
        def _(e):
            bits = plsc.load_gather(
                idxb,
                [jnp.full((16,), ewrow, jnp.int32),
                 jnp.full((16,), e, jnp.int32)],
            )
            ewb = plsc.bitcast(bits, jnp.float32)
            for f in range(D // 16):
                sl = pl.ds(f * 16, 16)
                sbuf[e, sl] = gbuf[e, sl] * ewb

    def gwait(gbuf, sem, ridx):
        pltpu.make_async_copy(hp_hbm.at[ridx], gbuf, sem).wait()

    def swait(sbuf, sem, cidx):
        pltpu.make_async_copy(sbuf, acc_sh.at[cidx], sem).wait()

    def idx_wait(sem):
        pltpu.make_async_copy(idx_hbm.at[wid, 0], idxb.at[pl.ds(0, 3)],
                              sem).wait()

    pltpu.async_copy(hp_hbm.at[idxb.at[0]], ga, semga)

    def pair(cp, carry):
        cha = 2 * cp
        chb = 2 * cp + 1
        slota = lax.rem(cha, 6)
        slotb = slota + 1
        slota2 = lax.rem(cha + 2, 6)
        spreva = lax.rem(cha + 4, 6)
        sprevb = spreva + 1

        @pl.when(cp > 0)
        def _():
            idx_wait(semib)

        pltpu.async_copy(hp_hbm.at[idxb.at[3 * slotb]], gb, semgb)
        gwait(ga, semga, idxb.at[3 * slota])

        @pl.when(cp > 0)
        def _():
            swait(sa, semsa, idxb.at[3 * spreva + 1])

        scale(ga, sa, 3 * slota + 2)
        pltpu.async_copy(sa, acc_sh.at[idxb.at[3 * slota + 1]], semsa,
                         add=True)

        @pl.when(jnp.logical_and(cp > 0, cp < NPAIR - 1))
        def _():
            idx_wait(semia)

        @pl.when(cp < NPAIR - 1)
        def _():
            pltpu.async_copy(hp_hbm.at[idxb.at[3 * slota2]], ga, semga)

        @pl.when(cp < NPAIR - 2)
        def _():
            pltpu.async_copy(idx_hbm.at[wid, cha + 4],
                             idxb.at[pl.ds(3 * spreva, 3)], semia)

        gwait(gb, semgb, idxb.at[3 * slotb])

        @pl.when(cp > 0)
        def _():
            swait(sb, semsb, idxb.at[3 * sprevb + 1])

        scale(gb, sb, 3 * slotb + 2)
        pltpu.async_copy(sb, acc_sh.at[idxb.at[3 * slotb + 1]], semsb,
                         add=True)

        @pl.when(cp < NPAIR - 1)
        def _():
            pltpu.async_copy(idx_hbm.at[wid, chb + 2],
                             idxb.at[pl.ds(3 * (slota2 + 1), 3)], semib)

        return carry

    lax.fori_loop(0, NPAIR, pair, 0)
    swait(sa, semsa, idxb.at[3 * ((NCHUNK - 2) % 6) + 1])
    swait(sb, semsb, idxb.at[3 * ((NCHUNK - 1) % 6) + 1])
    plsc.subcore_barrier()

    @pl.when(s < NS - 1)
    def _():
        pltpu.sync_copy(acc_sh.at[pl.ds(s * NROWS, NROWS)],
                        accp_hbm.at[c, pl.ds(s * NROWS, NROWS)])

    @pl.when(s == NS - 1)
    def _():
        pltpu.sync_copy(acc_sh.at[pl.ds((NS - 1) * NROWS, N - (NS - 1) * NROWS)],
                        accp_hbm.at[c, pl.ds((NS - 1) * NROWS, N - (NS - 1) * NROWS)])


def _final_body(accp_ref, hp_ref, dinv_ref, b_ref, out_ref):
    sacc = accp_ref[0] + accp_ref[1] + hp_ref[...]
    y = dinv_ref[...] * sacc + b_ref[...][None, :]
    out_ref[...] = jnp.where(y > 0, y, jnp.exp(y) - 1.0)


def _final_call(accp, hp, dinv, b1):
    return pl.pallas_call(
        _final_body,
        grid=(N // _MB,),
        in_specs=[
            pl.BlockSpec((NC, _MB, D), lambda i: (0, i, 0)),
            pl.BlockSpec((_MB, D), lambda i: (i, 0)),
            pl.BlockSpec((_MB, 1), lambda i: (i, 0)),
            pl.BlockSpec((D,), lambda i: (0,)),
        ],
        out_specs=pl.BlockSpec((_MB, D), lambda i: (i, 0)),
        out_shape=jax.ShapeDtypeStruct((N, D), jnp.float32),
    )(accp, hp, dinv, b1)


def kernel(x, edge_index, edge_weight, W0, b0, W1, b1):
    row = edge_index[0]
    col = edge_index[1]
    pad = E_PAD - E
    zi = jnp.zeros((pad,), jnp.int32)
    rowp = jnp.concatenate([row, zi])
    colp = jnp.concatenate([col, zi])
    ewp = jnp.concatenate([edge_weight, jnp.zeros((pad,), jnp.float32)])
    idx3 = jnp.stack(
        [rowp.reshape(NW, NCHUNK, K),
         colp.reshape(NW, NCHUNK, K),
         lax.bitcast_convert_type(ewp, jnp.int32).reshape(NW, NCHUNK, K)],
        axis=2)
    col3 = colp.reshape(NW, NCHUNK, K)
    ew3 = ewp.reshape(NW, NCHUNK, K)
    zn = jnp.zeros((N,), jnp.float32)
    znd = jnp.zeros((N, D), jnp.float32)

    degp = _deg_kernel(col3, ew3, zn)
    hp, dinv = _mid_call(x, W1, degp.T)
    accp = _scatter_kernel(idx3, hp, znd)
    return _final_call(accp, hp, dinv, b1)


# speedup vs baseline: 24.3010x; 1.0005x over previous
import functools

import jax
import jax.numpy as jnp
from jax import lax
from jax.experimental import pallas as pl
from jax.experimental.pallas import tpu as pltpu
from jax.experimental.pallas import tpu_sc as plsc

N = 10000
E = 320000
D = 128
NC = 2
NS = 16
NW = NC * NS
K = 80
NCHUNK = 126
EPW = NCHUNK * K
E_PAD = EPW * NW
NPAIR = NCHUNK // 2
NROWS = 624

_MESH = plsc.VectorSubcoreMesh(core_axis_name="c", subcore_axis_name="s")
_SC_PARAMS = pltpu.CompilerParams(needs_layout_passes=False)


_DEGW = 8


@functools.partial(
    pl.kernel,
    out_type=jax.ShapeDtypeStruct((NC, N), jnp.float32),
    mesh=_MESH,
    scratch_types=[
        pltpu.VMEM((NCHUNK, K), jnp.int32),
        pltpu.VMEM((NCHUNK, K), jnp.float32),
        pltpu.VMEM_SHARED((N,), jnp.float32),
        pltpu.SemaphoreType.DMA,
    ],
    compiler_params=_SC_PARAMS,
)
def _deg_kernel(col_hbm, ew_hbm, zn_hbm, degp_hbm, col_v, ew_v, deg_sh, sem):
    c = lax.axis_index("c")
    s = lax.axis_index("s")
    wid = s * NC + c

    @pl.when(s == 0)
    def _():
        pltpu.sync_copy(zn_hbm, deg_sh)

    pltpu.sync_copy(col_hbm.at[wid], col_v)
    pltpu.sync_copy(ew_hbm.at[wid], ew_v)
    plsc.subcore_barrier()

    def drain_one():
        pltpu.make_async_copy(ew_v.at[0], deg_sh.at[col_v.at[0]], sem).wait()

    def fire(ch, carry):
        pltpu.async_copy(ew_v.at[ch], deg_sh.at[col_v.at[ch]], sem, add=True)

        @pl.when(ch >= _DEGW)
        def _():
            drain_one()

        return carry

    lax.fori_loop(0, NCHUNK, fire, 0)

    def drain(ch, carry):
        drain_one()
        return carry

    lax.fori_loop(0, _DEGW, drain, 0)
    plsc.subcore_barrier()

    @pl.when(s == 0)
    def _():
        pltpu.sync_copy(deg_sh, degp_hbm.at[c])


_MB = 1000


def _mid_body(x_ref, w_ref, degp_ref, hp_ref, dinv_ref):
    deg = jnp.sum(degp_ref[...], axis=1) + 1.0
    dinv = lax.rsqrt(deg)
    h = jnp.dot(x_ref[...], w_ref[...], preferred_element_type=jnp.float32)
    hp_ref[...] = h * dinv[:, None]
    dinv_ref[...] = dinv[:, None]


def _mid_call(x, w1, degp_t):
    return pl.pallas_call(
        _mid_body,
        grid=(N // _MB,),
        in_specs=[
            pl.BlockSpec((_MB, D), lambda i: (i, 0)),
            pl.BlockSpec((D, D), lambda i: (0, 0)),
            pl.BlockSpec((_MB, NC), lambda i: (i, 0)),
        ],
        out_specs=[
            pl.BlockSpec((_MB, D), lambda i: (i, 0)),
            pl.BlockSpec((_MB, 1), lambda i: (i, 0)),
        ],
        out_shape=[
            jax.ShapeDtypeStruct((N, D), jnp.float32),
            jax.ShapeDtypeStruct((N, 1), jnp.float32),
        ],
    )(x, w1, degp_t)


@functools.partial(
    pl.kernel,
    out_type=jax.ShapeDtypeStruct((NC, N, D), jnp.float32),
    mesh=_MESH,
    scratch_types=[
        pltpu.VMEM((18, K), jnp.int32),
        pltpu.VMEM((K, D), jnp.float32),
        pltpu.VMEM((K, D), jnp.float32),
        pltpu.VMEM((K, D), jnp.float32),
        pltpu.VMEM((K, D), jnp.float32),
        pltpu.VMEM_SHARED((N, D), jnp.float32),
        pltpu.SemaphoreType.DMA,
        pltpu.SemaphoreType.DMA,
        pltpu.SemaphoreType.DMA,
        pltpu.SemaphoreType.DMA,
        pltpu.SemaphoreType.DMA,
        pltpu.SemaphoreType.DMA,
    ],
    compiler_params=_SC_PARAMS,
)
def _scatter_kernel(idx_hbm, hp_hbm, znd_hbm, accp_hbm,
                    idxb, ga, gb, sa, sb, acc_sh,
                    semga, semgb, semsa, semsb, semia, semib):
    c = lax.axis_index("c")
    s = lax.axis_index("s")
    wid = s * NC + c

    pltpu.sync_copy(idx_hbm.at[wid, 0], idxb.at[pl.ds(0, 3)])
    pltpu.sync_copy(idx_hbm.at[wid, 1], idxb.at[pl.ds(3, 3)])
    pltpu.sync_copy(idx_hbm.at[wid, 2], idxb.at[pl.ds(6, 3)])

    @pl.when(s < NS - 1)
    def _():
        pltpu.sync_copy(znd_hbm.at[pl.ds(s * NROWS, NROWS)],
                        acc_sh.at[pl.ds(s * NROWS, NROWS)])

    @pl.when(s == NS - 1)
    def _():
        pltpu.sync_copy(znd_hbm.at[pl.ds((NS - 1) * NROWS, N - (NS - 1) * NROWS)],
                        acc_sh.at[pl.ds((NS - 1) * NROWS, N - (NS - 1) * NROWS)])

    plsc.subcore_barrier()

    def scale(gbuf, sbuf, ewrow):
        @plsc.parallel_loop(0, K, unroll=8)
        def _(e):
            bits = plsc.load_gather(
                idxb,
                [jnp.full((16,), ewrow, jnp.int32),
                 jnp.full((16,), e, jnp.int32)],
            )
            ewb = plsc.bitcast(bits, jnp.float32)
            for f in range(D // 16):
                sl = pl.ds(f * 16, 16)
                sbuf[e, sl] = gbuf[e, sl] * ewb

    def gwait(gbuf, sem, ridx):
        pltpu.make_async_copy(hp_hbm.at[ridx], gbuf, sem).wait()

    def swait(sbuf, sem, cidx):
        pltpu.make_async_copy(sbuf, acc_sh.at[cidx], sem).wait()

    def idx_wait(sem):
        pltpu.make_async_copy(idx_hbm.at[wid, 0], idxb.at[pl.ds(0, 3)],
                              sem).wait()

    pltpu.async_copy(hp_hbm.at[idxb.at[0]], ga, semga)

    def pair(cp, carry):
        cha = 2 * cp
        chb = 2 * cp + 1
        slota = lax.rem(cha, 6)
        slotb = slota + 1
        slota2 = lax.rem(cha + 2, 6)
        spreva = lax.rem(cha + 4, 6)
        sprevb = spreva + 1

        @pl.when(cp > 0)
        def _():
            idx_wait(semib)

        pltpu.async_copy(hp_hbm.at[idxb.at[3 * slotb]], gb, semgb)
        gwait(ga, semga, idxb.at[3 * slota])

        @pl.when(cp > 0)
        def _():
            swait(sa, semsa, idxb.at[3 * spreva + 1])

        scale(ga, sa, 3 * slota + 2)
        pltpu.async_copy(sa, acc_sh.at[idxb.at[3 * slota + 1]], semsa,
                         add=True)

        @pl.when(jnp.logical_and(cp > 0, cp < NPAIR - 1))
        def _():
            idx_wait(semia)

        @pl.when(cp < NPAIR - 1)
        def _():
            pltpu.async_copy(hp_hbm.at[idxb.at[3 * slota2]], ga, semga)

        @pl.when(cp < NPAIR - 2)
        def _():
            pltpu.async_copy(idx_hbm.at[wid, cha + 4],
                             idxb.at[pl.ds(3 * spreva, 3)], semia)

        gwait(gb, semgb, idxb.at[3 * slotb])

        @pl.when(cp > 0)
        def _():
            swait(sb, semsb, idxb.at[3 * sprevb + 1])

        scale(gb, sb, 3 * slotb + 2)
        pltpu.async_copy(sb, acc_sh.at[idxb.at[3 * slotb + 1]], semsb,
                         add=True)

        @pl.when(cp < NPAIR - 1)
        def _():
            pltpu.async_copy(idx_hbm.at[wid, chb + 2],
                             idxb.at[pl.ds(3 * (slota2 + 1), 3)], semib)

        return carry

    lax.fori_loop(0, NPAIR, pair, 0)
    swait(sa, semsa, idxb.at[3 * ((NCHUNK - 2) % 6) + 1])
    swait(sb, semsb, idxb.at[3 * ((NCHUNK - 1) % 6) + 1])
    plsc.subcore_barrier()

    @pl.when(s < NS - 1)
    def _():
        pltpu.sync_copy(acc_sh.at[pl.ds(s * NROWS, NROWS)],
                        accp_hbm.at[c, pl.ds(s * NROWS, NROWS)])

    @pl.when(s == NS - 1)
    def _():
        pltpu.sync_copy(acc_sh.at[pl.ds((NS - 1) * NROWS, N - (NS - 1) * NROWS)],
                        accp_hbm.at[c, pl.ds((NS - 1) * NROWS, N - (NS - 1) * NROWS)])


def _final_body(accp_ref, hp_ref, dinv_ref, b_ref, out_ref):
    sacc = accp_ref[0] + accp_ref[1] + hp_ref[...]
    y = dinv_ref[...] * sacc + b_ref[...][None, :]
    out_ref[...] = jnp.where(y > 0, y, jnp.exp(y) - 1.0)


def _final_call(accp, hp, dinv, b1):
    return pl.pallas_call(
        _final_body,
        grid=(N // _MB,),
        in_specs=[
            pl.BlockSpec((NC, _MB, D), lambda i: (0, i, 0)),
            pl.BlockSpec((_MB, D), lambda i: (i, 0)),
            pl.BlockSpec((_MB, 1), lambda i: (i, 0)),
            pl.BlockSpec((D,), lambda i: (0,)),
        ],
        out_specs=pl.BlockSpec((_MB, D), lambda i: (i, 0)),
        out_shape=jax.ShapeDtypeStruct((N, D), jnp.float32),
    )(accp, hp, dinv, b1)


def kernel(x, edge_index, edge_weight, W0, b0, W1, b1):
    row = edge_index[0]
    col = edge_index[1]
    pad = E_PAD - E
    zi = jnp.zeros((pad,), jnp.int32)
    rowp = jnp.concatenate([row, zi])
    colp = jnp.concatenate([col, zi])
    ewp = jnp.concatenate([edge_weight, jnp.zeros((pad,), jnp.float32)])
    idx3 = jnp.stack(
        [rowp.reshape(NW, NCHUNK, K),
         colp.reshape(NW, NCHUNK, K),
         lax.bitcast_convert_type(ewp, jnp.int32).reshape(NW, NCHUNK, K)],
        axis=2)
    col3 = colp.reshape(NW, NCHUNK, K)
    ew3 = ewp.reshape(NW, NCHUNK, K)
    zn = jnp.zeros((N,), jnp.float32)
    znd = jnp.zeros((N, D), jnp.float32)

    degp = _deg_kernel(col3, ew3, zn)
    hp, dinv = _mid_call(x, W1, degp.T)
    accp = _scatter_kernel(idx3, hp, znd)
    return _final_call(accp, hp, dinv, b1)


# speedup vs baseline: 24.3039x; 1.0001x over previous
import functools

import jax
import jax.numpy as jnp
from jax import lax
from jax.experimental import pallas as pl
from jax.experimental.pallas import tpu as pltpu
from jax.experimental.pallas import tpu_sc as plsc

N = 10000
E = 320000
D = 128
NC = 2
NS = 16
NW = NC * NS
K = 80
NCHUNK = 126
EPW = NCHUNK * K
E_PAD = EPW * NW
NPAIR = NCHUNK // 2
NROWS = 624

_MESH = plsc.VectorSubcoreMesh(core_axis_name="c", subcore_axis_name="s")
_SC_PARAMS = pltpu.CompilerParams(needs_layout_passes=False)


_DEGW = 8


@functools.partial(
    pl.kernel,
    out_type=jax.ShapeDtypeStruct((NC, N), jnp.float32),
    mesh=_MESH,
    scratch_types=[
        pltpu.VMEM((NCHUNK, K), jnp.int32),
        pltpu.VMEM((NCHUNK, K), jnp.float32),
        pltpu.VMEM_SHARED((N,), jnp.float32),
        pltpu.SemaphoreType.DMA,
    ],
    compiler_params=_SC_PARAMS,
)
def _deg_kernel(col_hbm, ew_hbm, zn_hbm, degp_hbm, col_v, ew_v, deg_sh, sem):
    c = lax.axis_index("c")
    s = lax.axis_index("s")
    wid = s * NC + c

    @pl.when(s == 0)
    def _():
        pltpu.sync_copy(zn_hbm, deg_sh)

    pltpu.sync_copy(col_hbm.at[wid], col_v)
    pltpu.sync_copy(ew_hbm.at[wid], ew_v)
    plsc.subcore_barrier()

    def drain_one():
        pltpu.make_async_copy(ew_v.at[0], deg_sh.at[col_v.at[0]], sem).wait()

    def fire(ch, carry):
        pltpu.async_copy(ew_v.at[ch], deg_sh.at[col_v.at[ch]], sem, add=True)

        @pl.when(ch >= _DEGW)
        def _():
            drain_one()

        return carry

    lax.fori_loop(0, NCHUNK, fire, 0)

    def drain(ch, carry):
        drain_one()
        return carry

    lax.fori_loop(0, _DEGW, drain, 0)
    plsc.subcore_barrier()

    @pl.when(s == 0)
    def _():
        pltpu.sync_copy(deg_sh, degp_hbm.at[c])


_MB = 1000


def _mid_body(x_ref, w_ref, degp_ref, hp_ref, dinv_ref):
    deg = jnp.sum(degp_ref[...], axis=1) + 1.0
    dinv = lax.rsqrt(deg)
    h = jnp.dot(x_ref[...], w_ref[...], preferred_element_type=jnp.float32)
    hp_ref[...] = h * dinv[:, None]
    dinv_ref[...] = dinv[:, None]


def _mid_call(x, w1, degp_t):
    return pl.pallas_call(
        _mid_body,
        grid=(N // _MB,),
        in_specs=[
            pl.BlockSpec((_MB, D), lambda i: (i, 0)),
            pl.BlockSpec((D, D), lambda i: (0, 0)),
            pl.BlockSpec((_MB, NC), lambda i: (i, 0)),
        ],
        out_specs=[
            pl.BlockSpec((_MB, D), lambda i: (i, 0)),
            pl.BlockSpec((_MB, 1), lambda i: (i, 0)),
        ],
        out_shape=[
            jax.ShapeDtypeStruct((N, D), jnp.float32),
            jax.ShapeDtypeStruct((N, 1), jnp.float32),
        ],
    )(x, w1, degp_t)


@functools.partial(
    pl.kernel,
    out_type=jax.ShapeDtypeStruct((NC, N, D), jnp.float32),
    mesh=_MESH,
    scratch_types=[
        pltpu.VMEM((18, K), jnp.int32),
        pltpu.VMEM((K, D), jnp.float32),
        pltpu.VMEM((K, D), jnp.float32),
        pltpu.VMEM((K, D), jnp.float32),
        pltpu.VMEM((K, D), jnp.float32),
        pltpu.VMEM_SHARED((N, D), jnp.float32),
        pltpu.SemaphoreType.DMA,
        pltpu.SemaphoreType.DMA,
        pltpu.SemaphoreType.DMA,
        pltpu.SemaphoreType.DMA,
        pltpu.SemaphoreType.DMA,
        pltpu.SemaphoreType.DMA,
    ],
    compiler_params=_SC_PARAMS,
)
def _scatter_kernel(idx_hbm, hp_hbm, znd_hbm, accp_hbm,
                    idxb, ga, gb, sa, sb, acc_sh,
                    semga, semgb, semsa, semsb, semia, semib):
    c = lax.axis_index("c")
    s = lax.axis_index("s")
    wid = s * NC + c

    pltpu.sync_copy(idx_hbm.at[wid, 0], idxb.at[pl.ds(0, 3)])
    pltpu.sync_copy(idx_hbm.at[wid, 1], idxb.at[pl.ds(3, 3)])
    pltpu.sync_copy(idx_hbm.at[wid, 2], idxb.at[pl.ds(6, 3)])

    @pl.when(s < NS - 1)
    def _():
        pltpu.sync_copy(znd_hbm.at[pl.ds(s * NROWS, NROWS)],
                        acc_sh.at[pl.ds(s * NROWS, NROWS)])

    @pl.when(s == NS - 1)
    def _():
        pltpu.sync_copy(znd_hbm.at[pl.ds((NS - 1) * NROWS, N - (NS - 1) * NROWS)],
                        acc_sh.at[pl.ds((NS - 1) * NROWS, N - (NS - 1) * NROWS)])

    plsc.subcore_barrier()

    def scale(gbuf, sbuf, ewrow):
        @plsc.parallel_loop(0, K, unroll=4)
        def _(e):
            bits = plsc.load_gather(
                idxb,
                [jnp.full((16,), ewrow, jnp.int32),
                 jnp.full((16,), e, jnp.int32)],
            )
            ewb = plsc.bitcast(bits, jnp.float32)
            for f in range(D // 16):
                sl = pl.ds(f * 16, 16)
                sbuf[e, sl] = gbuf[e, sl] * ewb

    def gwait(gbuf, sem, ridx):
        pltpu.make_async_copy(hp_hbm.at[ridx], gbuf, sem).wait()

    def swait(sbuf, sem, cidx):
        pltpu.make_async_copy(sbuf, acc_sh.at[cidx], sem).wait()

    def idx_wait(sem):
        pltpu.make_async_copy(idx_hbm.at[wid, 0], idxb.at[pl.ds(0, 3)],
                              sem).wait()

    pltpu.async_copy(hp_hbm.at[idxb.at[0]], ga, semga)

    def pair(cp, carry):
        cha = 2 * cp
        chb = 2 * cp + 1
        slota = lax.rem(cha, 6)
        slotb = slota + 1
        slota2 = lax.rem(cha + 2, 6)
        spreva = lax.rem(cha + 4, 6)
        sprevb = spreva + 1

        @pl.when(cp > 0)
        def _():
            idx_wait(semib)

        pltpu.async_copy(hp_hbm.at[idxb.at[3 * slotb]], gb, semgb)
        gwait(ga, semga, idxb.at[3 * slota])

        @pl.when(cp > 0)
        def _():
            swait(sa, semsa, idxb.at[3 * spreva + 1])

        scale(ga, sa, 3 * slota + 2)
        pltpu.async_copy(sa, acc_sh.at[idxb.at[3 * slota + 1]], semsa,
                         add=True)

        @pl.when(jnp.logical_and(cp > 0, cp < NPAIR - 1))
        def _():
            idx_wait(semia)

        @pl.when(cp < NPAIR - 1)
        def _():
            pltpu.async_copy(hp_hbm.at[idxb.at[3 * slota2]], ga, semga)

        @pl.when(cp < NPAIR - 2)
        def _():
            pltpu.async_copy(idx_hbm.at[wid, cha + 4],
                             idxb.at[pl.ds(3 * spreva, 3)], semia)

        gwait(gb, semgb, idxb.at[3 * slotb])

        @pl.when(cp > 0)
        def _():
            swait(sb, semsb, idxb.at[3 * sprevb + 1])

        scale(gb, sb, 3 * slotb + 2)
        pltpu.async_copy(sb, acc_sh.at[idxb.at[3 * slotb + 1]], semsb,
                         add=True)

        @pl.when(cp < NPAIR - 1)
        def _():
            pltpu.async_copy(idx_hbm.at[wid, chb + 2],
                             idxb.at[pl.ds(3 * (slota2 + 1), 3)], semib)

        return carry

    lax.fori_loop(0, NPAIR, pair, 0)
    swait(sa, semsa, idxb.at[3 * ((NCHUNK - 2) % 6) + 1])
    swait(sb, semsb, idxb.at[3 * ((NCHUNK - 1) % 6) + 1])
    plsc.subcore_barrier()

    @pl.when(s < NS - 1)
    def _():
        pltpu.sync_copy(acc_sh.at[pl.ds(s * NROWS, NROWS)],
                        accp_hbm.at[c, pl.ds(s * NROWS, NROWS)])

    @pl.when(s == NS - 1)
    def _():
        pltpu.sync_copy(acc_sh.at[pl.ds((NS - 1) * NROWS, N - (NS - 1) * NROWS)],
                        accp_hbm.at[c, pl.ds((NS - 1) * NROWS, N - (NS - 1) * NROWS)])


def _final_body(accp_ref, hp_ref, dinv_ref, b_ref, out_ref):
    sacc = accp_ref[0] + accp_ref[1] + hp_ref[...]
    y = dinv_ref[...] * sacc + b_ref[...][None, :]
    out_ref[...] = jnp.where(y > 0, y, jnp.exp(y) - 1.0)


def _final_call(accp, hp, dinv, b1):
    return pl.pallas_call(
        _final_body,
        grid=(N // _MB,),
        in_specs=[
            pl.BlockSpec((NC, _MB, D), lambda i: (0, i, 0)),
            pl.BlockSpec((_MB, D), lambda i: (i, 0)),
            pl.BlockSpec((_MB, 1), lambda i: (i, 0)),
            pl.BlockSpec((D,), lambda i: (0,)),
        ],
        out_specs=pl.BlockSpec((_MB, D), lambda i: (i, 0)),
        out_shape=jax.ShapeDtypeStruct((N, D), jnp.float32),
    )(accp, hp, dinv, b1)


def kernel(x, edge_index, edge_weight, W0, b0, W1, b1):
    row = edge_index[0]
    col = edge_index[1]
    pad = E_PAD - E
    zi = jnp.zeros((pad,), jnp.int32)
    rowp = jnp.concatenate([row, zi])
    colp = jnp.concatenate([col, zi])
    ewp = jnp.concatenate([edge_weight, jnp.zeros((pad,), jnp.float32)])
    idx3 = jnp.stack(
        [rowp.reshape(NW, NCHUNK, K),
         colp.reshape(NW, NCHUNK, K),
         lax.bitcast_convert_type(ewp, jnp.int32).reshape(NW, NCHUNK, K)],
        axis=2)
    col3 = colp.reshape(NW, NCHUNK, K)
    ew3 = ewp.reshape(NW, NCHUNK, K)
    zn = jnp.zeros((N,), jnp.float32)
    znd = jnp.zeros((N, D), jnp.float32)

    degp = _deg_kernel(col3, ew3, zn)
    hp, dinv = _mid_call(x, W1, degp.T)
    accp = _scatter_kernel(idx3, hp, znd)
    return _final_call(accp, hp, dinv, b1)


# speedup vs baseline: 24.3056x; 1.0001x over previous
import functools

import jax
import jax.numpy as jnp
from jax import lax
from jax.experimental import pallas as pl
from jax.experimental.pallas import tpu as pltpu
from jax.experimental.pallas import tpu_sc as plsc

N = 10000
E = 320000
D = 128
NC = 2
NS = 16
NW = NC * NS
K = 80
NCHUNK = 126
EPW = NCHUNK * K
E_PAD = EPW * NW
NPAIR = NCHUNK // 2
NROWS = 624

_MESH = plsc.VectorSubcoreMesh(core_axis_name="c", subcore_axis_name="s",
                               num_cores=NC, num_subcores=NS)
_SC_PARAMS = pltpu.CompilerParams(needs_layout_passes=False)


_DEGW = 8


@functools.partial(
    pl.kernel,
    out_type=jax.ShapeDtypeStruct((NC, N), jnp.float32),
    mesh=_MESH,
    scratch_types=[
        pltpu.VMEM((NCHUNK, K), jnp.int32),
        pltpu.VMEM((NCHUNK, K), jnp.float32),
        pltpu.VMEM_SHARED((N,), jnp.float32),
        pltpu.SemaphoreType.DMA,
    ],
    compiler_params=_SC_PARAMS,
)
def _deg_kernel(col_hbm, ew_hbm, zn_hbm, degp_hbm, col_v, ew_v, deg_sh, sem):
    c = lax.axis_index("c")
    s = lax.axis_index("s")
    wid = s * NC + c

    @pl.when(s == 0)
    def _():
        pltpu.sync_copy(zn_hbm, deg_sh)

    pltpu.sync_copy(col_hbm.at[wid], col_v)
    pltpu.sync_copy(ew_hbm.at[wid], ew_v)
    plsc.subcore_barrier()

    def drain_one():
        pltpu.make_async_copy(ew_v.at[0], deg_sh.at[col_v.at[0]], sem).wait()

    def fire(ch, carry):
        pltpu.async_copy(ew_v.at[ch], deg_sh.at[col_v.at[ch]], sem, add=True)

        @pl.when(ch >= _DEGW)
        def _():
            drain_one()

        return carry

    lax.fori_loop(0, NCHUNK, fire, 0)

    def drain(ch, carry):
        drain_one()
        return carry

    lax.fori_loop(0, _DEGW, drain, 0)
    plsc.subcore_barrier()

    @pl.when(s == 0)
    def _():
        pltpu.sync_copy(deg_sh, degp_hbm.at[c])


_MB = 1000


def _mid_body(x_ref, w_ref, degp_ref, hp_ref, dinv_ref):
    deg = jnp.sum(degp_ref[...], axis=1) + 1.0
    dinv = lax.rsqrt(deg)
    h = jnp.dot(x_ref[...], w_ref[...], preferred_element_type=jnp.float32)
    hp_ref[...] = h * dinv[:, None]
    dinv_ref[...] = dinv[:, None]


def _mid_call(x, w1, degp_t):
    return pl.pallas_call(
        _mid_body,
        grid=(N // _MB,),
        in_specs=[
            pl.BlockSpec((_MB, D), lambda i: (i, 0)),
            pl.BlockSpec((D, D), lambda i: (0, 0)),
            pl.BlockSpec((_MB, NC), lambda i: (i, 0)),
        ],
        out_specs=[
            pl.BlockSpec((_MB, D), lambda i: (i, 0)),
            pl.BlockSpec((_MB, 1), lambda i: (i, 0)),
        ],
        out_shape=[
            jax.ShapeDtypeStruct((N, D), jnp.float32),
            jax.ShapeDtypeStruct((N, 1), jnp.float32),
        ],
    )(x, w1, degp_t)


@functools.partial(
    pl.kernel,
    out_type=jax.ShapeDtypeStruct((NC, N, D), jnp.float32),
    mesh=_MESH,
    scratch_types=[
        pltpu.VMEM((18, K), jnp.int32),
        pltpu.VMEM((K, D), jnp.float32),
        pltpu.VMEM((K, D), jnp.float32),
        pltpu.VMEM((K, D), jnp.float32),
        pltpu.VMEM((K, D), jnp.float32),
        pltpu.VMEM_SHARED((N, D), jnp.float32),
        pltpu.SemaphoreType.DMA,
        pltpu.SemaphoreType.DMA,
        pltpu.SemaphoreType.DMA,
        pltpu.SemaphoreType.DMA,
        pltpu.SemaphoreType.DMA,
        pltpu.SemaphoreType.DMA,
    ],
    compiler_params=_SC_PARAMS,
)
def _scatter_kernel(idx_hbm, hp_hbm, znd_hbm, accp_hbm,
                    idxb, ga, gb, sa, sb, acc_sh,
                    semga, semgb, semsa, semsb, semia, semib):
    c = lax.axis_index("c")
    s = lax.axis_index("s")
    wid = s * NC + c

    pltpu.sync_copy(idx_hbm.at[wid, 0], idxb.at[pl.ds(0, 3)])
    pltpu.sync_copy(idx_hbm.at[wid, 1], idxb.at[pl.ds(3, 3)])
    pltpu.sync_copy(idx_hbm.at[wid, 2], idxb.at[pl.ds(6, 3)])

    @pl.when(s < NS - 1)
    def _():
        pltpu.sync_copy(znd_hbm.at[pl.ds(s * NROWS, NROWS)],
                        acc_sh.at[pl.ds(s * NROWS, NROWS)])

    @pl.when(s == NS - 1)
    def _():
        pltpu.sync_copy(znd_hbm.at[pl.ds((NS - 1) * NROWS, N - (NS - 1) * NROWS)],
                        acc_sh.at[pl.ds((NS - 1) * NROWS, N - (NS - 1) * NROWS)])

    plsc.subcore_barrier()

    def scale(gbuf, sbuf, ewrow):
        @plsc.parallel_loop(0, K, unroll=4)
        def _(e):
            bits = plsc.load_gather(
                idxb,
                [jnp.full((16,), ewrow, jnp.int32),
                 jnp.full((16,), e, jnp.int32)],
            )
            ewb = plsc.bitcast(bits, jnp.float32)
            for f in range(D // 16):
                sl = pl.ds(f * 16, 16)
                sbuf[e, sl] = gbuf[e, sl] * ewb

    def gwait(gbuf, sem, ridx):
        pltpu.make_async_copy(hp_hbm.at[ridx], gbuf, sem).wait()

    def swait(sbuf, sem, cidx):
        pltpu.make_async_copy(sbuf, acc_sh.at[cidx], sem).wait()

    def idx_wait(sem):
        pltpu.make_async_copy(idx_hbm.at[wid, 0], idxb.at[pl.ds(0, 3)],
                              sem).wait()

    pltpu.async_copy(hp_hbm.at[idxb.at[0]], ga, semga)

    def pair(cp, carry):
        cha = 2 * cp
        chb = 2 * cp + 1
        slota = lax.rem(cha, 6)
        slotb = slota + 1
        slota2 = lax.rem(cha + 2, 6)
        spreva = lax.rem(cha + 4, 6)
        sprevb = spreva + 1

        @pl.when(cp > 0)
        def _():
            idx_wait(semib)

        pltpu.async_copy(hp_hbm.at[idxb.at[3 * slotb]], gb, semgb)
        gwait(ga, semga, idxb.at[3 * slota])

        @pl.when(cp > 0)
        def _():
            swait(sa, semsa, idxb.at[3 * spreva + 1])

        scale(ga, sa, 3 * slota + 2)
        pltpu.async_copy(sa, acc_sh.at[idxb.at[3 * slota + 1]], semsa,
                         add=True)

        @pl.when(jnp.logical_and(cp > 0, cp < NPAIR - 1))
        def _():
            idx_wait(semia)

        @pl.when(cp < NPAIR - 1)
        def _():
            pltpu.async_copy(hp_hbm.at[idxb.at[3 * slota2]], ga, semga)

        @pl.when(cp < NPAIR - 2)
        def _():
            pltpu.async_copy(idx_hbm.at[wid, cha + 4],
                             idxb.at[pl.ds(3 * spreva, 3)], semia)

        gwait(gb, semgb, idxb.at[3 * slotb])

        @pl.when(cp > 0)
        def _():
            swait(sb, semsb, idxb.at[3 * sprevb + 1])

        scale(gb, sb, 3 * slotb + 2)
        pltpu.async_copy(sb, acc_sh.at[idxb.at[3 * slotb + 1]], semsb,
                         add=True)

        @pl.when(cp < NPAIR - 1)
        def _():
            pltpu.async_copy(idx_hbm.at[wid, chb + 2],
                             idxb.at[pl.ds(3 * (slota2 + 1), 3)], semib)

        return carry

    lax.fori_loop(0, NPAIR, pair, 0)
    swait(sa, semsa, idxb.at[3 * ((NCHUNK - 2) % 6) + 1])
    swait(sb, semsb, idxb.at[3 * ((NCHUNK - 1) % 6) + 1])
    plsc.subcore_barrier()

    @pl.when(s < NS - 1)
    def _():
        pltpu.sync_copy(acc_sh.at[pl.ds(s * NROWS, NROWS)],
                        accp_hbm.at[c, pl.ds(s * NROWS, NROWS)])

    @pl.when(s == NS - 1)
    def _():
        pltpu.sync_copy(acc_sh.at[pl.ds((NS - 1) * NROWS, N - (NS - 1) * NROWS)],
                        accp_hbm.at[c, pl.ds((NS - 1) * NROWS, N - (NS - 1) * NROWS)])


def _final_body(accp_ref, hp_ref, dinv_ref, b_ref, out_ref):
    sacc = accp_ref[0] + accp_ref[1] + hp_ref[...]
    y = dinv_ref[...] * sacc + b_ref[...][None, :]
    out_ref[...] = jnp.where(y > 0, y, jnp.exp(y) - 1.0)


def _final_call(accp, hp, dinv, b1):
    return pl.pallas_call(
        _final_body,
        grid=(N // _MB,),
        in_specs=[
            pl.BlockSpec((NC, _MB, D), lambda i: (0, i, 0)),
            pl.BlockSpec((_MB, D), lambda i: (i, 0)),
            pl.BlockSpec((_MB, 1), lambda i: (i, 0)),
            pl.BlockSpec((D,), lambda i: (0,)),
        ],
        out_specs=pl.BlockSpec((_MB, D), lambda i: (i, 0)),
        out_shape=jax.ShapeDtypeStruct((N, D), jnp.float32),
    )(accp, hp, dinv, b1)


def kernel(x, edge_index, edge_weight, W0, b0, W1, b1):
    row = edge_index[0]
    col = edge_index[1]
    pad = E_PAD - E
    zi = jnp.zeros((pad,), jnp.int32)
    rowp = jnp.concatenate([row, zi])
    colp = jnp.concatenate([col, zi])
    ewp = jnp.concatenate([edge_weight, jnp.zeros((pad,), jnp.float32)])
    idx3 = jnp.stack(
        [rowp.reshape(NW, NCHUNK, K),
         colp.reshape(NW, NCHUNK, K),
         lax.bitcast_convert_type(ewp, jnp.int32).reshape(NW, NCHUNK, K)],
        axis=2)
    col3 = colp.reshape(NW, NCHUNK, K)
    ew3 = ewp.reshape(NW, NCHUNK, K)
    zn = jnp.zeros((N,), jnp.float32)
    znd = jnp.zeros((N, D), jnp.float32)

    degp = _deg_kernel(col3, ew3, zn)
    hp, dinv = _mid_call(x, W1, degp.T)
    accp = _scatter_kernel(idx3, hp, znd)
    return _final_call(accp, hp, dinv, b1)
